# initial kernel scaffold (unmeasured)
import jax
import jax.numpy as jnp
from jax import lax
from jax.experimental import pallas as pl
from jax.experimental.pallas import tpu as pltpu

N_Z = 4
M = 2048
D = 2048
CHUNK = M // N_Z


def kernel(partial, resid, gamma):
    gamma2d = gamma.reshape(1, D)

    def body(partial_ref, resid_ref, gamma_ref, out_ref,
             rs_buf, rs_send_sems, rs_recv_sems, ag_send_sems, ag_recv_sems):
        my_x = lax.axis_index("x")
        my_y = lax.axis_index("y")
        r = lax.axis_index("z")
        right = (r + 1) % N_Z
        left = (r + 3) % N_Z

        barrier_sem = pltpu.get_barrier_semaphore()
        for nbr in (left, right):
            pl.semaphore_signal(
                barrier_sem, inc=1,
                device_id=(my_x, my_y, nbr),
                device_id_type=pl.DeviceIdType.MESH,
            )
        pl.semaphore_wait(barrier_sem, 2)

        for s in range(N_Z - 1):
            if s == 0:
                src = partial_ref.at[0, pl.ds(r * CHUNK, CHUNK), :]
            else:
                src = rs_buf.at[s - 1]
            rdma = pltpu.make_async_remote_copy(
                src_ref=src,
                dst_ref=rs_buf.at[s],
                send_sem=rs_send_sems.at[s],
                recv_sem=rs_recv_sems.at[s],
                device_id=(my_x, my_y, right),
                device_id_type=pl.DeviceIdType.MESH,
            )
            rdma.start()
            rdma.wait()
            c = (r + (N_Z - 1 - s)) % N_Z
            rs_buf[s, :, :] = (
                rs_buf[s, :, :] + partial_ref[0, pl.ds(c * CHUNK, CHUNK), :]
            )

        q = (r + 1) % N_Z
        y = rs_buf[N_Z - 2, :, :] + resid_ref[pl.ds(q * CHUNK, CHUNK), :]
        ms = jnp.mean(y * y, axis=1, keepdims=True)
        out_ref[pl.ds(q * CHUNK, CHUNK), :] = (
            y * lax.rsqrt(ms + 1e-6) * gamma_ref[0, :]
        )

        for s in range(N_Z - 1):
            c = (r + (N_Z + 1 - s)) % N_Z
            sl = pl.ds(c * CHUNK, CHUNK)
            rdma = pltpu.make_async_remote_copy(
                src_ref=out_ref.at[sl, :],
                dst_ref=out_ref.at[sl, :],
                send_sem=ag_send_sems.at[s],
                recv_sem=ag_recv_sems.at[s],
                device_id=(my_x, my_y, right),
                device_id_type=pl.DeviceIdType.MESH,
            )
            rdma.start()
            rdma.wait()

    return pl.pallas_call(
        body,
        out_shape=jax.ShapeDtypeStruct((M, D), jnp.float32),
        in_specs=[
            pl.BlockSpec(memory_space=pltpu.VMEM),
            pl.BlockSpec(memory_space=pltpu.VMEM),
            pl.BlockSpec(memory_space=pltpu.VMEM),
        ],
        out_specs=pl.BlockSpec(memory_space=pltpu.VMEM),
        scratch_shapes=[
            pltpu.VMEM((N_Z - 1, CHUNK, D), jnp.float32),
            pltpu.SemaphoreType.DMA((N_Z - 1,)),
            pltpu.SemaphoreType.DMA((N_Z - 1,)),
            pltpu.SemaphoreType.DMA((N_Z - 1,)),
            pltpu.SemaphoreType.DMA((N_Z - 1,)),
        ],
        compiler_params=pltpu.CompilerParams(collective_id=0),
    )(partial, resid, gamma2d)


# baseline (device time: 311776 ns/iter reference)
import jax
import jax.numpy as jnp
from jax import lax
from jax.experimental import pallas as pl
from jax.experimental.pallas import tpu as pltpu

N_Z = 4
M = 2048
D = 2048
CHUNK = M // N_Z


def kernel(partial, resid, gamma):
    gamma2d = gamma.reshape(1, D)

    def body(partial_ref, resid_ref, gamma_ref, out_ref,
             rs_buf, resid_chunk, rs_send_sems, rs_recv_sems,
             ag_send_sems, ag_recv_sems, resid_sem):
        my_x = lax.axis_index("x")
        my_y = lax.axis_index("y")
        r = lax.axis_index("z")
        right = (r + 1) % N_Z
        left = (r + 3) % N_Z

        barrier_sem = pltpu.get_barrier_semaphore()
        for nbr in (left, right):
            pl.semaphore_signal(
                barrier_sem, inc=1,
                device_id=(my_x, my_y, nbr),
                device_id_type=pl.DeviceIdType.MESH,
            )
        pl.semaphore_wait(barrier_sem, 2)

        q = (r + 1) % N_Z
        resid_copy = pltpu.make_async_copy(
            resid_ref.at[pl.ds(q * CHUNK, CHUNK), :], resid_chunk, resid_sem
        )
        resid_copy.start()

        for s in range(N_Z - 1):
            if s == 0:
                src = partial_ref.at[0, pl.ds(r * CHUNK, CHUNK), :]
            else:
                src = rs_buf.at[s - 1]
            rdma = pltpu.make_async_remote_copy(
                src_ref=src,
                dst_ref=rs_buf.at[s],
                send_sem=rs_send_sems.at[s],
                recv_sem=rs_recv_sems.at[s],
                device_id=(my_x, my_y, right),
                device_id_type=pl.DeviceIdType.MESH,
            )
            rdma.start()
            rdma.wait()
            c = (r + (N_Z - 1 - s)) % N_Z
            rs_buf[s, :, :] = (
                rs_buf[s, :, :] + partial_ref[0, pl.ds(c * CHUNK, CHUNK), :]
            )

        resid_copy.wait()
        y = rs_buf[N_Z - 2, :, :] + resid_chunk[:, :]
        ms = jnp.mean(y * y, axis=1, keepdims=True)
        out_ref[pl.ds(q * CHUNK, CHUNK), :] = (
            y * lax.rsqrt(ms + 1e-6) * gamma_ref[0, :]
        )

        for s in range(N_Z - 1):
            c = (r + (N_Z + 1 - s)) % N_Z
            sl = pl.ds(c * CHUNK, CHUNK)
            rdma = pltpu.make_async_remote_copy(
                src_ref=out_ref.at[sl, :],
                dst_ref=out_ref.at[sl, :],
                send_sem=ag_send_sems.at[s],
                recv_sem=ag_recv_sems.at[s],
                device_id=(my_x, my_y, right),
                device_id_type=pl.DeviceIdType.MESH,
            )
            rdma.start()
            rdma.wait()

    return pl.pallas_call(
        body,
        out_shape=jax.ShapeDtypeStruct((M, D), jnp.float32),
        in_specs=[
            pl.BlockSpec(memory_space=pltpu.VMEM),
            pl.BlockSpec(memory_space=pltpu.MemorySpace.HBM),
            pl.BlockSpec(memory_space=pltpu.VMEM),
        ],
        out_specs=pl.BlockSpec(memory_space=pltpu.VMEM),
        scratch_shapes=[
            pltpu.VMEM((N_Z - 1, CHUNK, D), jnp.float32),
            pltpu.VMEM((CHUNK, D), jnp.float32),
            pltpu.SemaphoreType.DMA((N_Z - 1,)),
            pltpu.SemaphoreType.DMA((N_Z - 1,)),
            pltpu.SemaphoreType.DMA((N_Z - 1,)),
            pltpu.SemaphoreType.DMA((N_Z - 1,)),
            pltpu.SemaphoreType.DMA,
        ],
        compiler_params=pltpu.CompilerParams(
            collective_id=0, vmem_limit_bytes=100 * 1024 * 1024
        ),
    )(partial, resid, gamma2d)


# device time: 310646 ns/iter; 1.0036x vs baseline; 1.0036x over previous
import jax
import jax.numpy as jnp
from jax import lax
from jax.experimental import pallas as pl
from jax.experimental.pallas import tpu as pltpu

N_Z = 4
M = 2048
D = 2048
HALF = M // 2
CHUNK = HALF // N_Z


def kernel(partial, resid, gamma):
    gamma2d = gamma.reshape(1, D)

    def body(partial_ref, resid_ref, gamma_ref, out_ref,
             rs_bufA, rs_bufB, resid_chunk,
             send_sems, rsA_recv, rsB_recv, agA_recv, agB_recv, resid_sems):
        my_x = lax.axis_index("x")
        my_y = lax.axis_index("y")
        r = lax.axis_index("z")
        right = (r + 1) % N_Z
        left = (r + 3) % N_Z

        barrier_sem = pltpu.get_barrier_semaphore()
        for nbr in (left, right):
            pl.semaphore_signal(
                barrier_sem, inc=1,
                device_id=(my_x, my_y, nbr),
                device_id_type=pl.DeviceIdType.MESH,
            )
        pl.semaphore_wait(barrier_sem, 2)

        qA = (r + 1) % N_Z
        qB = (r + 3) % N_Z
        rcopyA = pltpu.make_async_copy(
            resid_ref.at[pl.ds(qA * CHUNK, CHUNK), :],
            resid_chunk.at[0], resid_sems.at[0],
        )
        rcopyB = pltpu.make_async_copy(
            resid_ref.at[pl.ds(HALF + qB * CHUNK, CHUNK), :],
            resid_chunk.at[1], resid_sems.at[1],
        )
        rcopyA.start()
        rcopyB.start()

        for s in range(N_Z - 1):
            if s == 0:
                srcA = partial_ref.at[0, pl.ds(r * CHUNK, CHUNK), :]
                srcB = partial_ref.at[0, pl.ds(HALF + r * CHUNK, CHUNK), :]
            else:
                srcA = rs_bufA.at[s - 1]
                srcB = rs_bufB.at[s - 1]
            rdmaA = pltpu.make_async_remote_copy(
                src_ref=srcA,
                dst_ref=rs_bufA.at[s],
                send_sem=send_sems.at[0],
                recv_sem=rsA_recv.at[s],
                device_id=(my_x, my_y, right),
                device_id_type=pl.DeviceIdType.MESH,
            )
            rdmaB = pltpu.make_async_remote_copy(
                src_ref=srcB,
                dst_ref=rs_bufB.at[s],
                send_sem=send_sems.at[1],
                recv_sem=rsB_recv.at[s],
                device_id=(my_x, my_y, left),
                device_id_type=pl.DeviceIdType.MESH,
            )
            rdmaA.start()
            rdmaB.start()
            rdmaA.wait()
            cA = (r + (N_Z - 1 - s)) % N_Z
            rs_bufA[s, :, :] = (
                rs_bufA[s, :, :] + partial_ref[0, pl.ds(cA * CHUNK, CHUNK), :]
            )
            rdmaB.wait()
            cB = (r + s + 1) % N_Z
            rs_bufB[s, :, :] = (
                rs_bufB[s, :, :]
                + partial_ref[0, pl.ds(HALF + cB * CHUNK, CHUNK), :]
            )

        rcopyA.wait()
        yA = rs_bufA[N_Z - 2, :, :] + resid_chunk[0, :, :]
        msA = jnp.mean(yA * yA, axis=1, keepdims=True)
        out_ref[pl.ds(qA * CHUNK, CHUNK), :] = (
            yA * lax.rsqrt(msA + 1e-6) * gamma_ref[0, :]
        )
        rcopyB.wait()
        yB = rs_bufB[N_Z - 2, :, :] + resid_chunk[1, :, :]
        msB = jnp.mean(yB * yB, axis=1, keepdims=True)
        out_ref[pl.ds(HALF + qB * CHUNK, CHUNK), :] = (
            yB * lax.rsqrt(msB + 1e-6) * gamma_ref[0, :]
        )

        for s in range(N_Z - 1):
            cA = (r + (N_Z + 1 - s)) % N_Z
            cB = (r + 3 + s) % N_Z
            slA = pl.ds(cA * CHUNK, CHUNK)
            slB = pl.ds(HALF + cB * CHUNK, CHUNK)
            rdmaA = pltpu.make_async_remote_copy(
                src_ref=out_ref.at[slA, :],
                dst_ref=out_ref.at[slA, :],
                send_sem=send_sems.at[2],
                recv_sem=agA_recv.at[s],
                device_id=(my_x, my_y, right),
                device_id_type=pl.DeviceIdType.MESH,
            )
            rdmaB = pltpu.make_async_remote_copy(
                src_ref=out_ref.at[slB, :],
                dst_ref=out_ref.at[slB, :],
                send_sem=send_sems.at[3],
                recv_sem=agB_recv.at[s],
                device_id=(my_x, my_y, left),
                device_id_type=pl.DeviceIdType.MESH,
            )
            rdmaA.start()
            rdmaB.start()
            rdmaA.wait()
            rdmaB.wait()

    return pl.pallas_call(
        body,
        out_shape=jax.ShapeDtypeStruct((M, D), jnp.float32),
        in_specs=[
            pl.BlockSpec(memory_space=pltpu.VMEM),
            pl.BlockSpec(memory_space=pltpu.MemorySpace.HBM),
            pl.BlockSpec(memory_space=pltpu.VMEM),
        ],
        out_specs=pl.BlockSpec(memory_space=pltpu.VMEM),
        scratch_shapes=[
            pltpu.VMEM((N_Z - 1, CHUNK, D), jnp.float32),
            pltpu.VMEM((N_Z - 1, CHUNK, D), jnp.float32),
            pltpu.VMEM((2, CHUNK, D), jnp.float32),
            pltpu.SemaphoreType.DMA((4,)),
            pltpu.SemaphoreType.DMA((N_Z - 1,)),
            pltpu.SemaphoreType.DMA((N_Z - 1,)),
            pltpu.SemaphoreType.DMA((N_Z - 1,)),
            pltpu.SemaphoreType.DMA((N_Z - 1,)),
            pltpu.SemaphoreType.DMA((2,)),
        ],
        compiler_params=pltpu.CompilerParams(
            collective_id=0, vmem_limit_bytes=100 * 1024 * 1024
        ),
    )(partial, resid, gamma2d)


# device time: 246911 ns/iter; 1.2627x vs baseline; 1.2581x over previous
import jax
import jax.numpy as jnp
from jax import lax
from jax.experimental import pallas as pl
from jax.experimental.pallas import tpu as pltpu

N_Z = 4
M = 2048
D = 2048
QTR = M // 4
CHUNK = QTR // N_Z
HALF = M // 2


def kernel(partial, resid, gamma):
    gamma2d = gamma.reshape(1, D)

    def body(partial_ref, resid_ref, gamma_ref, out_ref,
             rs_buf, resid_chunk,
             rs_send, rs_recv, ag_send, ag_recv,
             y_send, y_recv, x_send, x_recv, resid_sem):
        my_x = lax.axis_index("x")
        my_y = lax.axis_index("y")
        r = lax.axis_index("z")
        right = (r + 1) % N_Z
        left = (r + 3) % N_Z
        q = 2 * my_x + my_y
        qbase = q * QTR

        barrier_sem = pltpu.get_barrier_semaphore()
        for dev in ((my_x, my_y, left), (my_x, my_y, right),
                    (my_x, 1 - my_y, r), (1 - my_x, my_y, r)):
            pl.semaphore_signal(
                barrier_sem, inc=1,
                device_id=dev, device_id_type=pl.DeviceIdType.MESH,
            )
        pl.semaphore_wait(barrier_sem, 4)

        o = (r + 1) % N_Z
        obase = qbase + o * CHUNK
        rcopy = pltpu.make_async_copy(
            resid_ref.at[pl.ds(obase, CHUNK), :], resid_chunk, resid_sem
        )
        rcopy.start()

        for s in range(N_Z - 1):
            if s == 0:
                src = partial_ref.at[0, pl.ds(qbase + r * CHUNK, CHUNK), :]
            else:
                src = rs_buf.at[s - 1]
            rdma = pltpu.make_async_remote_copy(
                src_ref=src,
                dst_ref=rs_buf.at[s],
                send_sem=rs_send,
                recv_sem=rs_recv.at[s],
                device_id=(my_x, my_y, right),
                device_id_type=pl.DeviceIdType.MESH,
            )
            rdma.start()
            rdma.wait()
            c = (r + (N_Z - 1 - s)) % N_Z
            rs_buf[s, :, :] = (
                rs_buf[s, :, :]
                + partial_ref[0, pl.ds(qbase + c * CHUNK, CHUNK), :]
            )

        rcopy.wait()
        y = rs_buf[N_Z - 2, :, :] + resid_chunk[:, :]
        ms = jnp.mean(y * y, axis=1, keepdims=True)
        out_ref[pl.ds(obase, CHUNK), :] = (
            y * lax.rsqrt(ms + 1e-6) * gamma_ref[0, :]
        )

        for s in range(N_Z - 1):
            c = (r + (N_Z + 1 - s)) % N_Z
            sl = pl.ds(qbase + c * CHUNK, CHUNK)
            rdma = pltpu.make_async_remote_copy(
                src_ref=out_ref.at[sl, :],
                dst_ref=out_ref.at[sl, :],
                send_sem=ag_send,
                recv_sem=ag_recv.at[s],
                device_id=(my_x, my_y, right),
                device_id_type=pl.DeviceIdType.MESH,
            )
            rdma.start()
            rdma.wait()

        sl = pl.ds(qbase, QTR)
        rdma = pltpu.make_async_remote_copy(
            src_ref=out_ref.at[sl, :],
            dst_ref=out_ref.at[sl, :],
            send_sem=y_send,
            recv_sem=y_recv,
            device_id=(my_x, 1 - my_y, r),
            device_id_type=pl.DeviceIdType.MESH,
        )
        rdma.start()
        rdma.wait()

        sl = pl.ds(my_x * HALF, HALF)
        rdma = pltpu.make_async_remote_copy(
            src_ref=out_ref.at[sl, :],
            dst_ref=out_ref.at[sl, :],
            send_sem=x_send,
            recv_sem=x_recv,
            device_id=(1 - my_x, my_y, r),
            device_id_type=pl.DeviceIdType.MESH,
        )
        rdma.start()
        rdma.wait()

    return pl.pallas_call(
        body,
        out_shape=jax.ShapeDtypeStruct((M, D), jnp.float32),
        in_specs=[
            pl.BlockSpec(memory_space=pltpu.VMEM),
            pl.BlockSpec(memory_space=pltpu.MemorySpace.HBM),
            pl.BlockSpec(memory_space=pltpu.VMEM),
        ],
        out_specs=pl.BlockSpec(memory_space=pltpu.VMEM),
        scratch_shapes=[
            pltpu.VMEM((N_Z - 1, CHUNK, D), jnp.float32),
            pltpu.VMEM((CHUNK, D), jnp.float32),
            pltpu.SemaphoreType.DMA,
            pltpu.SemaphoreType.DMA((N_Z - 1,)),
            pltpu.SemaphoreType.DMA,
            pltpu.SemaphoreType.DMA((N_Z - 1,)),
            pltpu.SemaphoreType.DMA,
            pltpu.SemaphoreType.DMA,
            pltpu.SemaphoreType.DMA,
            pltpu.SemaphoreType.DMA,
            pltpu.SemaphoreType.DMA,
        ],
        compiler_params=pltpu.CompilerParams(
            collective_id=0, vmem_limit_bytes=100 * 1024 * 1024
        ),
    )(partial, resid, gamma2d)


# device time: 162981 ns/iter; 1.9130x vs baseline; 1.5150x over previous
import jax
import jax.numpy as jnp
from jax import lax
from jax.experimental import pallas as pl
from jax.experimental.pallas import tpu as pltpu

N_Z = 4
M = 2048
D = 2048
QTR = M // 4
CHUNK = QTR // N_Z
HALF = M // 2


def kernel(partial, resid, gamma):
    gamma2d = gamma.reshape(1, D)

    def body(partial_ref, resid_ref, gamma_ref, out_ref,
             rs_buf, resid_chunk,
             rs_send, rs_recv, ag_send, ag_recv,
             y_send, y_recv, x_send, x_recv, resid_sem):
        my_x = lax.axis_index("x")
        my_y = lax.axis_index("y")
        r = lax.axis_index("z")
        right = (r + 1) % N_Z
        left = (r + 3) % N_Z
        q = 2 * my_x + my_y
        qbase = q * QTR

        barrier_sem = pltpu.get_barrier_semaphore()
        for dev in ((my_x, my_y, left), (my_x, my_y, right),
                    (my_x, 1 - my_y, r), (1 - my_x, my_y, r)):
            pl.semaphore_signal(
                barrier_sem, inc=1,
                device_id=dev, device_id_type=pl.DeviceIdType.MESH,
            )
        pl.semaphore_wait(barrier_sem, 4)

        o = (r + 1) % N_Z
        obase = qbase + o * CHUNK
        rcopy = pltpu.make_async_copy(
            resid_ref.at[pl.ds(obase, CHUNK), :], resid_chunk, resid_sem
        )
        rcopy.start()

        for s in range(N_Z - 1):
            if s == 0:
                src = partial_ref.at[0, pl.ds(qbase + r * CHUNK, CHUNK), :]
            else:
                src = rs_buf.at[s - 1]
            rdma = pltpu.make_async_remote_copy(
                src_ref=src,
                dst_ref=rs_buf.at[s],
                send_sem=rs_send,
                recv_sem=rs_recv.at[s],
                device_id=(my_x, my_y, right),
                device_id_type=pl.DeviceIdType.MESH,
            )
            rdma.start()
            rdma.wait()
            c = (r + (N_Z - 1 - s)) % N_Z
            rs_buf[s, :, :] = (
                rs_buf[s, :, :]
                + partial_ref[0, pl.ds(qbase + c * CHUNK, CHUNK), :]
            )

        rcopy.wait()
        y = rs_buf[N_Z - 2, :, :] + resid_chunk[:, :]
        ms = jnp.mean(y * y, axis=1, keepdims=True)
        out_ref[pl.ds(obase, CHUNK), :] = (
            y * lax.rsqrt(ms + 1e-6) * gamma_ref[0, :]
        )

        y_dev = (my_x, 1 - my_y, r)
        x_dev = (1 - my_x, my_y, r)
        qpbase = (2 * my_x + (1 - my_y)) * QTR

        def remote(sl, send, recv, dev):
            return pltpu.make_async_remote_copy(
                src_ref=out_ref.at[sl, :],
                dst_ref=out_ref.at[sl, :],
                send_sem=send,
                recv_sem=recv,
                device_id=dev,
                device_id_type=pl.DeviceIdType.MESH,
            )

        pending = []
        x_descs = []

        def send_x(sl):
            k = len(x_descs)
            d = remote(sl, x_send.at[k], x_recv.at[k], x_dev)
            d.start()
            x_descs.append(d)
            pending.append(d)

        sl_own = pl.ds(obase, CHUNK)
        ag = remote(sl_own, ag_send.at[0], ag_recv.at[0],
                    (my_x, my_y, right))
        ag.start()
        ag_descs = [ag]
        yd = remote(sl_own, y_send.at[0], y_recv.at[0], y_dev)
        yd.start()
        y_descs = [yd]
        pending.append(yd)
        send_x(sl_own)

        for s in range(N_Z - 1):
            ag_descs[s].wait_recv()
            pending.append(ag_descs[s])
            nb = (r + N_Z - s) % N_Z
            sl_nb = pl.ds(qbase + nb * CHUNK, CHUNK)
            if s < N_Z - 2:
                ag = remote(sl_nb, ag_send.at[s + 1], ag_recv.at[s + 1],
                            (my_x, my_y, right))
                ag.start()
                ag_descs.append(ag)
            yd = remote(sl_nb, y_send.at[s + 1], y_recv.at[s + 1], y_dev)
            yd.start()
            y_descs.append(yd)
            pending.append(yd)
            send_x(sl_nb)
            y_descs[s].wait_recv()
            yb = (r + N_Z + 1 - s) % N_Z
            send_x(pl.ds(qpbase + yb * CHUNK, CHUNK))

        y_descs[N_Z - 1].wait_recv()
        yb = (r + 2) % N_Z
        send_x(pl.ds(qpbase + yb * CHUNK, CHUNK))

        for d in pending:
            d.wait_send()
        for d in x_descs:
            d.wait_recv()

    return pl.pallas_call(
        body,
        out_shape=jax.ShapeDtypeStruct((M, D), jnp.float32),
        in_specs=[
            pl.BlockSpec(memory_space=pltpu.VMEM),
            pl.BlockSpec(memory_space=pltpu.MemorySpace.HBM),
            pl.BlockSpec(memory_space=pltpu.VMEM),
        ],
        out_specs=pl.BlockSpec(memory_space=pltpu.VMEM),
        scratch_shapes=[
            pltpu.VMEM((N_Z - 1, CHUNK, D), jnp.float32),
            pltpu.VMEM((CHUNK, D), jnp.float32),
            pltpu.SemaphoreType.DMA,
            pltpu.SemaphoreType.DMA((N_Z - 1,)),
            pltpu.SemaphoreType.DMA((N_Z - 1,)),
            pltpu.SemaphoreType.DMA((N_Z - 1,)),
            pltpu.SemaphoreType.DMA((N_Z,)),
            pltpu.SemaphoreType.DMA((N_Z,)),
            pltpu.SemaphoreType.DMA((2 * N_Z,)),
            pltpu.SemaphoreType.DMA((2 * N_Z,)),
            pltpu.SemaphoreType.DMA,
        ],
        compiler_params=pltpu.CompilerParams(
            collective_id=0, vmem_limit_bytes=100 * 1024 * 1024
        ),
    )(partial, resid, gamma2d)


# device time: 158638 ns/iter; 1.9653x vs baseline; 1.0274x over previous
import jax
import jax.numpy as jnp
from jax import lax
from jax.experimental import pallas as pl
from jax.experimental.pallas import tpu as pltpu

N_Z = 4
M = 2048
D = 2048
QTR = M // 4
CHUNK = QTR // N_Z
HALF = M // 2


def kernel(partial, resid, gamma):
    gamma2d = gamma.reshape(1, D)

    def body(partial_ref, resid_ref, gamma_ref, out_ref,
             rs_buf, resid_chunk,
             rs_send, rs_recv, ag_send, ag_recv,
             y_send, y_recv, x_send, x_recv, resid_sem):
        my_x = lax.axis_index("x")
        my_y = lax.axis_index("y")
        r = lax.axis_index("z")
        right = (r + 1) % N_Z
        left = (r + 3) % N_Z
        q = 2 * my_x + my_y
        qbase = q * QTR

        barrier_sem = pltpu.get_barrier_semaphore()
        for dev in ((my_x, my_y, left), (my_x, my_y, right),
                    (my_x, 1 - my_y, r), (1 - my_x, my_y, r)):
            pl.semaphore_signal(
                barrier_sem, inc=1,
                device_id=dev, device_id_type=pl.DeviceIdType.MESH,
            )
        pl.semaphore_wait(barrier_sem, 4)

        o = (r + 1) % N_Z
        obase = qbase + o * CHUNK
        rcopy = pltpu.make_async_copy(
            resid_ref.at[pl.ds(obase, CHUNK), :], resid_chunk, resid_sem
        )
        rcopy.start()

        SUB = CHUNK // 2

        def rs_rdma(s, h, src):
            return pltpu.make_async_remote_copy(
                src_ref=src,
                dst_ref=rs_buf.at[s, pl.ds(h * SUB, SUB), :],
                send_sem=rs_send.at[s, h],
                recv_sem=rs_recv.at[s, h],
                device_id=(my_x, my_y, right),
                device_id_type=pl.DeviceIdType.MESH,
            )

        pending = []
        rs_descs = {}
        for h in range(2):
            d = rs_rdma(
                0, h,
                partial_ref.at[0, pl.ds(qbase + r * CHUNK + h * SUB, SUB), :],
            )
            d.start()
            rs_descs[(0, h)] = d
        for s in range(N_Z - 1):
            c = (r + (N_Z - 1 - s)) % N_Z
            for h in range(2):
                d = rs_descs[(s, h)]
                d.wait_recv()
                pending.append(d)
                rs_buf[s, pl.ds(h * SUB, SUB), :] = (
                    rs_buf[s, pl.ds(h * SUB, SUB), :]
                    + partial_ref[
                        0, pl.ds(qbase + c * CHUNK + h * SUB, SUB), :
                    ]
                )
                if s < N_Z - 2:
                    nd = rs_rdma(
                        s + 1, h, rs_buf.at[s, pl.ds(h * SUB, SUB), :]
                    )
                    nd.start()
                    rs_descs[(s + 1, h)] = nd

        rcopy.wait()
        y = rs_buf[N_Z - 2, :, :] + resid_chunk[:, :]
        ms = jnp.mean(y * y, axis=1, keepdims=True)
        out_ref[pl.ds(obase, CHUNK), :] = (
            y * lax.rsqrt(ms + 1e-6) * gamma_ref[0, :]
        )

        y_dev = (my_x, 1 - my_y, r)
        x_dev = (1 - my_x, my_y, r)
        qpbase = (2 * my_x + (1 - my_y)) * QTR

        def remote(sl, send, recv, dev):
            return pltpu.make_async_remote_copy(
                src_ref=out_ref.at[sl, :],
                dst_ref=out_ref.at[sl, :],
                send_sem=send,
                recv_sem=recv,
                device_id=dev,
                device_id_type=pl.DeviceIdType.MESH,
            )

        x_descs = []

        def send_x(sl):
            k = len(x_descs)
            d = remote(sl, x_send.at[k], x_recv.at[k], x_dev)
            d.start()
            x_descs.append(d)
            pending.append(d)

        sl_own = pl.ds(obase, CHUNK)
        ag = remote(sl_own, ag_send.at[0], ag_recv.at[0],
                    (my_x, my_y, right))
        ag.start()
        ag_descs = [ag]
        yd = remote(sl_own, y_send.at[0], y_recv.at[0], y_dev)
        yd.start()
        y_descs = [yd]
        pending.append(yd)
        send_x(sl_own)

        for s in range(N_Z - 1):
            ag_descs[s].wait_recv()
            pending.append(ag_descs[s])
            nb = (r + N_Z - s) % N_Z
            sl_nb = pl.ds(qbase + nb * CHUNK, CHUNK)
            if s < N_Z - 2:
                ag = remote(sl_nb, ag_send.at[s + 1], ag_recv.at[s + 1],
                            (my_x, my_y, right))
                ag.start()
                ag_descs.append(ag)
            yd = remote(sl_nb, y_send.at[s + 1], y_recv.at[s + 1], y_dev)
            yd.start()
            y_descs.append(yd)
            pending.append(yd)
            send_x(sl_nb)
            y_descs[s].wait_recv()
            yb = (r + N_Z + 1 - s) % N_Z
            send_x(pl.ds(qpbase + yb * CHUNK, CHUNK))

        y_descs[N_Z - 1].wait_recv()
        yb = (r + 2) % N_Z
        send_x(pl.ds(qpbase + yb * CHUNK, CHUNK))

        for d in pending:
            d.wait_send()
        for d in x_descs:
            d.wait_recv()

    return pl.pallas_call(
        body,
        out_shape=jax.ShapeDtypeStruct((M, D), jnp.float32),
        in_specs=[
            pl.BlockSpec(memory_space=pltpu.VMEM),
            pl.BlockSpec(memory_space=pltpu.MemorySpace.HBM),
            pl.BlockSpec(memory_space=pltpu.VMEM),
        ],
        out_specs=pl.BlockSpec(memory_space=pltpu.VMEM),
        scratch_shapes=[
            pltpu.VMEM((N_Z - 1, CHUNK, D), jnp.float32),
            pltpu.VMEM((CHUNK, D), jnp.float32),
            pltpu.SemaphoreType.DMA((N_Z - 1, 2)),
            pltpu.SemaphoreType.DMA((N_Z - 1, 2)),
            pltpu.SemaphoreType.DMA((N_Z - 1,)),
            pltpu.SemaphoreType.DMA((N_Z - 1,)),
            pltpu.SemaphoreType.DMA((N_Z,)),
            pltpu.SemaphoreType.DMA((N_Z,)),
            pltpu.SemaphoreType.DMA((2 * N_Z,)),
            pltpu.SemaphoreType.DMA((2 * N_Z,)),
            pltpu.SemaphoreType.DMA,
        ],
        compiler_params=pltpu.CompilerParams(
            collective_id=0, vmem_limit_bytes=100 * 1024 * 1024
        ),
    )(partial, resid, gamma2d)


# device time: 137745 ns/iter; 2.2634x vs baseline; 1.1517x over previous
import jax
import jax.numpy as jnp
from jax import lax
from jax.experimental import pallas as pl
from jax.experimental.pallas import tpu as pltpu

N_Z = 4
M = 2048
D = 2048
QTR = M // 4
CHUNK = QTR // N_Z
HALF = M // 2


def kernel(partial, resid, gamma):
    gamma2d = gamma.reshape(1, D)

    def body(partial_ref, resid_ref, gamma_ref, out_ref,
             rs_buf, resid_chunk,
             rs_send, rs_recv, ag_send, ag_recv,
             y_send, y_recv, x_send, x_recv, resid_sem):
        my_x = lax.axis_index("x")
        my_y = lax.axis_index("y")
        r = lax.axis_index("z")
        right = (r + 1) % N_Z
        left = (r + 3) % N_Z
        q = 2 * my_x + my_y
        qbase = q * QTR

        barrier_sem = pltpu.get_barrier_semaphore()
        for dev in ((my_x, my_y, left), (my_x, my_y, right),
                    (my_x, 1 - my_y, r), (1 - my_x, my_y, r)):
            pl.semaphore_signal(
                barrier_sem, inc=1,
                device_id=dev, device_id_type=pl.DeviceIdType.MESH,
            )
        pl.semaphore_wait(barrier_sem, 4)

        o = (r + 1) % N_Z
        obase = qbase + o * CHUNK
        rcopy = pltpu.make_async_copy(
            resid_ref.at[pl.ds(obase, CHUNK), :], resid_chunk, resid_sem
        )
        rcopy.start()

        SUB = CHUNK // 2

        def rs_rdma(s, h, src):
            return pltpu.make_async_remote_copy(
                src_ref=src,
                dst_ref=rs_buf.at[s, pl.ds(h * SUB, SUB), :],
                send_sem=rs_send.at[s, h],
                recv_sem=rs_recv.at[s, h],
                device_id=(my_x, my_y, right),
                device_id_type=pl.DeviceIdType.MESH,
            )

        pending = []
        rs_descs = {}
        for h in range(2):
            d = rs_rdma(
                0, h,
                partial_ref.at[0, pl.ds(qbase + r * CHUNK + h * SUB, SUB), :],
            )
            d.start()
            rs_descs[(0, h)] = d
        for s in range(N_Z - 1):
            c = (r + (N_Z - 1 - s)) % N_Z
            for h in range(2):
                d = rs_descs[(s, h)]
                d.wait_recv()
                pending.append(d)
                rs_buf[s, pl.ds(h * SUB, SUB), :] = (
                    rs_buf[s, pl.ds(h * SUB, SUB), :]
                    + partial_ref[
                        0, pl.ds(qbase + c * CHUNK + h * SUB, SUB), :
                    ]
                )
                if s < N_Z - 2:
                    nd = rs_rdma(
                        s + 1, h, rs_buf.at[s, pl.ds(h * SUB, SUB), :]
                    )
                    nd.start()
                    rs_descs[(s + 1, h)] = nd

        rcopy.wait()
        y = rs_buf[N_Z - 2, :, :] + resid_chunk[:, :]
        ms = jnp.mean(y * y, axis=1, keepdims=True)
        out_ref[pl.ds(obase, CHUNK), :] = (
            y * lax.rsqrt(ms + 1e-6) * gamma_ref[0, :]
        )

        y_dev = (my_x, 1 - my_y, r)
        x_dev = (1 - my_x, my_y, r)
        qpbase = (2 * my_x + (1 - my_y)) * QTR
        qxbase = (2 * (1 - my_x) + my_y) * QTR

        def remote(sl, send, recv, dev):
            return pltpu.make_async_remote_copy(
                src_ref=out_ref.at[sl, :],
                dst_ref=out_ref.at[sl, :],
                send_sem=send,
                recv_sem=recv,
                device_id=dev,
                device_id_type=pl.DeviceIdType.MESH,
            )

        x_descs = {}
        y_descs = {}

        def send_x(slot, sl):
            d = remote(sl, x_send.at[slot], x_recv.at[slot], x_dev)
            d.start()
            x_descs[slot] = d
            pending.append(d)

        def send_y(slot, sl):
            d = remote(sl, y_send.at[slot], y_recv.at[slot], y_dev)
            d.start()
            y_descs[slot] = d
            pending.append(d)

        sl_own = pl.ds(obase, CHUNK)
        ag = remote(sl_own, ag_send.at[0], ag_recv.at[0],
                    (my_x, my_y, right))
        ag.start()
        ag_descs = [ag]
        send_y(0, sl_own)
        send_x(0, sl_own)

        for s in range(N_Z - 1):
            ag_descs[s].wait_recv()
            pending.append(ag_descs[s])
            nb = (r + N_Z - s) % N_Z
            sl_nb = pl.ds(qbase + nb * CHUNK, CHUNK)
            if s < N_Z - 2:
                ag = remote(sl_nb, ag_send.at[s + 1], ag_recv.at[s + 1],
                            (my_x, my_y, right))
                ag.start()
                ag_descs.append(ag)
            send_y(s + 1, sl_nb)
            send_x(2 * (s + 1), sl_nb)
            y_descs[s].wait_recv()
            yb = (r + N_Z + 1 - s) % N_Z
            send_x(2 * s + 1, pl.ds(qpbase + yb * CHUNK + SUB, SUB))

        y_descs[N_Z - 1].wait_recv()
        yb = (r + 2) % N_Z
        send_x(2 * (N_Z - 1) + 1, pl.ds(qpbase + yb * CHUNK + SUB, SUB))

        for k in range(N_Z):
            x_descs[2 * k].wait_recv()
            xb = (r + N_Z + 1 - k) % N_Z
            send_y(N_Z + k, pl.ds(qxbase + xb * CHUNK, SUB))

        for d in pending:
            d.wait_send()
        for k in range(N_Z):
            x_descs[2 * k + 1].wait_recv()
            y_descs[N_Z + k].wait_recv()

    return pl.pallas_call(
        body,
        out_shape=jax.ShapeDtypeStruct((M, D), jnp.float32),
        in_specs=[
            pl.BlockSpec(memory_space=pltpu.VMEM),
            pl.BlockSpec(memory_space=pltpu.MemorySpace.HBM),
            pl.BlockSpec(memory_space=pltpu.VMEM),
        ],
        out_specs=pl.BlockSpec(memory_space=pltpu.VMEM),
        scratch_shapes=[
            pltpu.VMEM((N_Z - 1, CHUNK, D), jnp.float32),
            pltpu.VMEM((CHUNK, D), jnp.float32),
            pltpu.SemaphoreType.DMA((N_Z - 1, 2)),
            pltpu.SemaphoreType.DMA((N_Z - 1, 2)),
            pltpu.SemaphoreType.DMA((N_Z - 1,)),
            pltpu.SemaphoreType.DMA((N_Z - 1,)),
            pltpu.SemaphoreType.DMA((2 * N_Z,)),
            pltpu.SemaphoreType.DMA((2 * N_Z,)),
            pltpu.SemaphoreType.DMA((2 * N_Z,)),
            pltpu.SemaphoreType.DMA((2 * N_Z,)),
            pltpu.SemaphoreType.DMA,
        ],
        compiler_params=pltpu.CompilerParams(
            collective_id=0, vmem_limit_bytes=100 * 1024 * 1024
        ),
    )(partial, resid, gamma2d)
